# baseline (speedup 1.0000x reference)
import jax
import jax.numpy as jnp
import numpy as np
from jax.experimental import pallas as pl
from jax.experimental.pallas import tpu as pltpu

_BINS = 10
_MOMENTUM = 0.5
_TILE = 1024


def _ghm_tile_kernel(ii_ref, jj_ref, xr_ref, xc_ref, tr_ref, tc_ref,
                     out_ref, acc_ref):
    p = pl.program_id(0)
    npairs = pl.num_programs(0)

    @pl.when(p == 0)
    def _init():
        for k in range(2 * _BINS):
            acc_ref[k] = jnp.float32(0.0)

    same = ii_ref[p] == jj_ref[p]
    w = jnp.where(same, jnp.float32(1.0), jnp.float32(2.0))

    xr = xr_ref[...]
    xc = xc_ref[...]
    rn = jnp.sqrt(jnp.sum(xr * xr, axis=1, keepdims=True))
    cn = jnp.sqrt(jnp.sum(xc * xc, axis=1, keepdims=True))
    xrn = xr / jnp.maximum(rn, 1e-12)
    xcn = xc / jnp.maximum(cn, 1e-12)
    cos = jax.lax.dot_general(
        xrn, xcn, (((1,), (1,)), ((), ())),
        preferred_element_type=jnp.float32)
    label = (tr_ref[...] == tc_ref[...]).astype(jnp.float32)
    g = jnp.abs(cos - label)
    g2 = g * g

    edges = [np.float32(i / _BINS) for i in range(_BINS + 1)]
    edges[-1] = np.float32(1.0 + 1e-6)
    for k in range(_BINS):
        m = g < edges[k + 1]
        acc_ref[k] += w * jnp.sum(m.astype(jnp.float32))
        acc_ref[_BINS + k] += w * jnp.sum(jnp.where(m, g2, 0.0))

    @pl.when(p == npairs - 1)
    def _finish():
        valid_num = acc_ref[_BINS - 1]
        loss = jnp.float32(0.0)
        prev_c = jnp.float32(0.0)
        prev_s = jnp.float32(0.0)
        for k in range(_BINS):
            c_k = acc_ref[k] - prev_c
            s_k = acc_ref[_BINS + k] - prev_s
            prev_c = acc_ref[k]
            prev_s = acc_ref[_BINS + k]
            loss = loss + s_k / ((1.0 - _MOMENTUM) * c_k + 1e-6)
        out_ref[0, 0] = jnp.where(valid_num > 0.0, loss / valid_num, 0.0)


@jax.jit
def kernel(x, targets):
    b = x.shape[0]
    xf = x.reshape(b, -1)
    nb = b // _TILE
    pairs = [(i, j) for i in range(nb) for j in range(i, nb)]
    ii = jnp.array([p[0] for p in pairs], dtype=jnp.int32)
    jj = jnp.array([p[1] for p in pairs], dtype=jnp.int32)
    npairs = len(pairs)

    t_col = targets.reshape(b, 1)
    t_row = targets.reshape(1, b)

    grid_spec = pltpu.PrefetchScalarGridSpec(
        num_scalar_prefetch=2,
        grid=(npairs,),
        in_specs=[
            pl.BlockSpec((_TILE, xf.shape[1]), lambda p, ii, jj: (ii[p], 0)),
            pl.BlockSpec((_TILE, xf.shape[1]), lambda p, ii, jj: (jj[p], 0)),
            pl.BlockSpec((_TILE, 1), lambda p, ii, jj: (ii[p], 0)),
            pl.BlockSpec((1, _TILE), lambda p, ii, jj: (0, jj[p])),
        ],
        out_specs=pl.BlockSpec((1, 1), lambda p, ii, jj: (0, 0),
                               memory_space=pltpu.SMEM),
        scratch_shapes=[pltpu.SMEM((2 * _BINS,), jnp.float32)],
    )
    loss = pl.pallas_call(
        _ghm_tile_kernel,
        grid_spec=grid_spec,
        out_shape=jax.ShapeDtypeStruct((1, 1), jnp.float32),
    )(ii, jj, xf, xf, t_col, t_row)
    return loss[0, 0]


# speedup vs baseline: 1.0359x; 1.0359x over previous
import jax
import jax.numpy as jnp
import numpy as np
from jax.experimental import pallas as pl
from jax.experimental.pallas import tpu as pltpu

_BINS = 10
_MOMENTUM = 0.5
_TILE = 512


def _ghm_tile_kernel(ii_ref, jj_ref, xr_ref, xc_ref, tr_ref, tc_ref,
                     out_ref, acc_ref):
    p = pl.program_id(0)
    npairs = pl.num_programs(0)

    @pl.when(p == 0)
    def _init():
        for k in range(2 * _BINS):
            acc_ref[k] = jnp.float32(0.0)

    same = ii_ref[p] == jj_ref[p]
    w = jnp.where(same, jnp.float32(1.0), jnp.float32(2.0))

    xr = xr_ref[...]
    xc = xc_ref[...]
    rn = jnp.sqrt(jnp.sum(xr * xr, axis=1, keepdims=True))
    cn = jnp.sqrt(jnp.sum(xc * xc, axis=1, keepdims=True))
    xrn = xr / jnp.maximum(rn, 1e-12)
    xcn = xc / jnp.maximum(cn, 1e-12)
    cos = jax.lax.dot_general(
        xrn, xcn, (((1,), (1,)), ((), ())),
        preferred_element_type=jnp.float32)
    label = (tr_ref[...] == tc_ref[...]).astype(jnp.float32)
    g = jnp.abs(cos - label)
    g2 = g * g

    edges = [np.float32(i / _BINS) for i in range(_BINS + 1)]
    edges[-1] = np.float32(1.0 + 1e-6)
    for k in range(_BINS):
        m = g < edges[k + 1]
        acc_ref[k] += w * jnp.sum(m.astype(jnp.float32))
        acc_ref[_BINS + k] += w * jnp.sum(jnp.where(m, g2, 0.0))

    @pl.when(p == npairs - 1)
    def _finish():
        valid_num = acc_ref[_BINS - 1]
        loss = jnp.float32(0.0)
        prev_c = jnp.float32(0.0)
        prev_s = jnp.float32(0.0)
        for k in range(_BINS):
            c_k = acc_ref[k] - prev_c
            s_k = acc_ref[_BINS + k] - prev_s
            prev_c = acc_ref[k]
            prev_s = acc_ref[_BINS + k]
            loss = loss + s_k / ((1.0 - _MOMENTUM) * c_k + 1e-6)
        out_ref[0, 0] = jnp.where(valid_num > 0.0, loss / valid_num, 0.0)


@jax.jit
def kernel(x, targets):
    b = x.shape[0]
    xf = x.reshape(b, -1)
    nb = b // _TILE
    pairs = [(i, j) for i in range(nb) for j in range(i, nb)]
    ii = jnp.array([p[0] for p in pairs], dtype=jnp.int32)
    jj = jnp.array([p[1] for p in pairs], dtype=jnp.int32)
    npairs = len(pairs)

    t_col = targets.reshape(b, 1)
    t_row = targets.reshape(1, b)

    grid_spec = pltpu.PrefetchScalarGridSpec(
        num_scalar_prefetch=2,
        grid=(npairs,),
        in_specs=[
            pl.BlockSpec((_TILE, xf.shape[1]), lambda p, ii, jj: (ii[p], 0)),
            pl.BlockSpec((_TILE, xf.shape[1]), lambda p, ii, jj: (jj[p], 0)),
            pl.BlockSpec((_TILE, 1), lambda p, ii, jj: (ii[p], 0)),
            pl.BlockSpec((1, _TILE), lambda p, ii, jj: (0, jj[p])),
        ],
        out_specs=pl.BlockSpec((1, 1), lambda p, ii, jj: (0, 0),
                               memory_space=pltpu.SMEM),
        scratch_shapes=[pltpu.SMEM((2 * _BINS,), jnp.float32)],
    )
    loss = pl.pallas_call(
        _ghm_tile_kernel,
        grid_spec=grid_spec,
        out_shape=jax.ShapeDtypeStruct((1, 1), jnp.float32),
    )(ii, jj, xf, xf, t_col, t_row)
    return loss[0, 0]
